# baseline (speedup 1.0000x reference)
import numpy as np
import jax
import jax.numpy as jnp
from jax.experimental import pallas as pl
from jax.experimental.pallas import tpu as pltpu

_N = 384
_H = 193
_HP = 200
_K = 16
_S = 2


def _dft_consts():
    j = np.arange(_N)
    m = np.outer(j, j) % _N
    ang = -2.0 * np.pi * m / _N
    fre = (np.cos(ang) / np.sqrt(_N)).astype(np.float32)
    fim = (np.sin(ang) / np.sqrt(_N)).astype(np.float32)
    fre_h = np.zeros((_HP, _N), np.float32)
    fim_h = np.zeros((_HP, _N), np.float32)
    fre_h[:_H] = fre[:_H]
    fim_h[:_H] = fim[:_H]
    import ml_dtypes
    bf = ml_dtypes.bfloat16
    return fre_h.astype(bf), fim_h.astype(bf), fre.astype(bf), fim.astype(bf)


_FRE_H, _FIM_H, _FRE, _FIM = _dft_consts()


def _fft_topk_kernel(x_ref, freh_ref, fimh_ref, fre_ref, fim_ref, out_ref):
    freh = freh_ref[...]
    fimh = fimh_ref[...]
    fre = fre_ref[...]
    fim = fim_ref[...]

    def dot(a, b):
        return jax.lax.dot(a, b, preferred_element_type=jnp.float32)

    neg = jnp.float32(-np.inf)
    row_iota = jax.lax.broadcasted_iota(jnp.int32, (_HP, _N), 0)
    wfull = jnp.where((row_iota == 0) | (row_iota == _H - 1),
                      jnp.float32(1.0), jnp.float32(2.0))
    lane = jax.lax.broadcasted_iota(jnp.int32, (1, _K), 1).astype(jnp.float32)

    carry = []
    for s in range(_S):
        x = x_ref[s]
        bre = dot(freh, x)
        bim = dot(fimh, x)
        breb = bre.astype(jnp.bfloat16)
        bimb = bim.astype(jnp.bfloat16)
        yre = dot(breb, fre) - dot(bimb, fim)
        yim = dot(breb, fim) + dot(bimb, fre)
        p = yre * yre + yim * yim
        carry.append((p, jnp.zeros((1, _K), jnp.float32),
                      jnp.zeros((1, 1), jnp.float32)))

    def body(_, carry):
        new = []
        for s in range(_S):
            p, out, cnt = carry[s]
            m = jnp.max(p, axis=(0, 1), keepdims=True)
            eq = p == m
            w = jnp.sum(jnp.where(eq, wfull, jnp.float32(0.0)),
                        axis=(0, 1), keepdims=True)
            p = jnp.where(eq, neg, p)
            val = jnp.sqrt(m)
            out = jnp.where((lane >= cnt) & (lane < cnt + w), val, out)
            new.append((p, out, cnt + w))
        return tuple(new)

    carry = jax.lax.fori_loop(0, _K, body, tuple(carry))
    for s in range(_S):
        out_ref[s] = carry[s][1]


def kernel(mtrx):
    b, c, h, w = mtrx.shape
    x = mtrx.reshape(b * c, h, w).astype(jnp.bfloat16)
    out = pl.pallas_call(
        _fft_topk_kernel,
        grid=(b * c // _S,),
        in_specs=[
            pl.BlockSpec((_S, h, w), lambda i: (i, 0, 0)),
            pl.BlockSpec((_HP, _N), lambda i: (0, 0)),
            pl.BlockSpec((_HP, _N), lambda i: (0, 0)),
            pl.BlockSpec((_N, _N), lambda i: (0, 0)),
            pl.BlockSpec((_N, _N), lambda i: (0, 0)),
        ],
        out_specs=pl.BlockSpec((_S, 1, _K), lambda i: (i, 0, 0)),
        out_shape=jax.ShapeDtypeStruct((b * c, 1, _K), jnp.float32),
        compiler_params=pltpu.CompilerParams(
            dimension_semantics=("arbitrary",)),
    )(x, jnp.asarray(_FRE_H), jnp.asarray(_FIM_H),
      jnp.asarray(_FRE), jnp.asarray(_FIM))
    return out.reshape(b, c, _K)
